# initial kernel scaffold (unmeasured)
import functools
import math

import jax
import jax.numpy as jnp
from jax import lax
from jax.experimental import pallas as pl
from jax.experimental.pallas import tpu as pltpu

N_DEV = 16
QBLK = 512


def kernel(q, k, v):
    s_per, d = q.shape
    scale = 1.0 / math.sqrt(d)
    n_qblk = s_per // QBLK

    q = q.astype(jnp.bfloat16)
    k = k.astype(jnp.bfloat16)
    v = v.astype(jnp.bfloat16)

    def body(q_ref, k_ref, v_ref, out_ref,
             kv_ref, acc_ref, m_ref, l_ref,
             send_sems, recv_sems, credit_sem):
        my = lax.axis_index("i")
        left = lax.rem(my + N_DEV - 1, N_DEV)
        right = lax.rem(my + 1, N_DEV)

        barrier = pltpu.get_barrier_semaphore()
        for nbr in (left, right):
            pl.semaphore_signal(
                barrier, inc=1,
                device_id=(nbr,), device_id_type=pl.DeviceIdType.MESH,
            )
        pl.semaphore_wait(barrier, 2)

        kv_ref[0, 0] = k_ref[...]
        kv_ref[0, 1] = v_ref[...]
        m_ref[...] = jnp.full((s_per, 1), -jnp.inf, jnp.float32)
        l_ref[...] = jnp.zeros((s_per, 1), jnp.float32)
        acc_ref[...] = jnp.zeros((s_per, d), jnp.float32)

        def qblock(b, cur):
            r = pl.ds(b * QBLK, QBLK)
            qb = q_ref[r, :]
            k_cur = kv_ref[cur, 0]
            v_cur = kv_ref[cur, 1]
            s = lax.dot_general(
                qb, k_cur, (((1,), (1,)), ((), ())),
                preferred_element_type=jnp.float32,
            ) * scale
            m_old = m_ref[r, :]
            m_new = jnp.maximum(m_old, jnp.max(s, axis=1, keepdims=True))
            p = jnp.exp(s - m_new)
            corr = jnp.exp(m_old - m_new)
            l_ref[r, :] = l_ref[r, :] * corr + jnp.sum(p, axis=1, keepdims=True)
            acc_ref[r, :] = acc_ref[r, :] * corr + jnp.dot(
                p.astype(jnp.bfloat16), v_cur,
                preferred_element_type=jnp.float32,
            )
            m_ref[r, :] = m_new
            return cur

        for h in range(N_DEV):
            cur = h % 2
            nxt = 1 - cur
            if h < N_DEV - 1:
                if h >= 1:
                    pl.semaphore_wait(credit_sem, 1)
                rdma = pltpu.make_async_remote_copy(
                    src_ref=kv_ref.at[cur],
                    dst_ref=kv_ref.at[nxt],
                    send_sem=send_sems.at[cur],
                    recv_sem=recv_sems.at[nxt],
                    device_id=(right,),
                    device_id_type=pl.DeviceIdType.MESH,
                )
                rdma.start()
            lax.fori_loop(0, n_qblk, qblock, cur)
            if h < N_DEV - 1:
                rdma.wait()
                if h <= N_DEV - 3:
                    pl.semaphore_signal(
                        credit_sem, inc=1,
                        device_id=(left,), device_id_type=pl.DeviceIdType.MESH,
                    )

        out_ref[...] = acc_ref[...] / l_ref[...]

    try:
        params = pltpu.CompilerParams(collective_id=0)
    except AttributeError:
        params = pltpu.TPUCompilerParams(collective_id=0)

    return pl.pallas_call(
        body,
        out_shape=jax.ShapeDtypeStruct((s_per, d), jnp.float32),
        in_specs=[pl.BlockSpec(memory_space=pltpu.VMEM)] * 3,
        out_specs=pl.BlockSpec(memory_space=pltpu.VMEM),
        scratch_shapes=[
            pltpu.VMEM((2, 2, s_per, d), jnp.bfloat16),
            pltpu.VMEM((s_per, d), jnp.float32),
            pltpu.VMEM((s_per, 1), jnp.float32),
            pltpu.VMEM((s_per, 1), jnp.float32),
            pltpu.SemaphoreType.DMA((2,)),
            pltpu.SemaphoreType.DMA((2,)),
            pltpu.SemaphoreType.REGULAR,
        ],
        compiler_params=params,
    )(q, k, v)


# baseline (device time: 1510157 ns/iter reference)
import functools
import math

import jax
import jax.numpy as jnp
from jax import lax
from jax.experimental import pallas as pl
from jax.experimental.pallas import tpu as pltpu

N_DEV = 16
QBLK = 512


def kernel(q, k, v):
    s_per, d = q.shape
    scale = 1.0 / math.sqrt(d)
    n_qblk = s_per // QBLK

    q = q.astype(jnp.bfloat16)
    k = k.astype(jnp.bfloat16)
    v = v.astype(jnp.bfloat16)

    def body(q_ref, k_ref, v_ref, out_ref,
             kv_ref, acc_ref, m_ref, l_ref,
             send_sems, recv_sems, credit_sem):
        my = lax.axis_index("i")
        left = lax.rem(my + N_DEV - 1, N_DEV)
        right = lax.rem(my + 1, N_DEV)

        barrier = pltpu.get_barrier_semaphore()
        for nbr in (left, right):
            pl.semaphore_signal(
                barrier, inc=1,
                device_id=(nbr,), device_id_type=pl.DeviceIdType.MESH,
            )
        pl.semaphore_wait(barrier, 2)

        kv_ref[0, 0] = k_ref[...]
        kv_ref[0, 1] = v_ref[...]
        m_ref[...] = jnp.full((s_per, 1), -jnp.inf, jnp.float32)
        l_ref[...] = jnp.zeros((s_per, 1), jnp.float32)
        acc_ref[...] = jnp.zeros((s_per, d), jnp.float32)

        def qblock(b, cur):
            r = pl.ds(b * QBLK, QBLK)
            qb = q_ref[r, :]
            k_cur = kv_ref[cur, 0]
            v_cur = kv_ref[cur, 1]
            s = lax.dot_general(
                qb, k_cur, (((1,), (1,)), ((), ())),
                preferred_element_type=jnp.float32,
            ) * scale
            m_old = m_ref[r, :]
            m_new = jnp.maximum(m_old, jnp.max(s, axis=1, keepdims=True))
            p = jnp.exp(s - m_new)
            corr = jnp.exp(m_old - m_new)
            l_ref[r, :] = l_ref[r, :] * corr + jnp.sum(p, axis=1, keepdims=True)
            acc_ref[r, :] = acc_ref[r, :] * corr + jnp.dot(
                p.astype(jnp.bfloat16), v_cur,
                preferred_element_type=jnp.float32,
            )
            m_ref[r, :] = m_new
            return cur

        for h in range(N_DEV):
            cur = h % 2
            nxt = 1 - cur
            if h < N_DEV - 1:
                if h >= 1:
                    pl.semaphore_wait(credit_sem, 1)
                rdma = pltpu.make_async_remote_copy(
                    src_ref=kv_ref.at[cur],
                    dst_ref=kv_ref.at[nxt],
                    send_sem=send_sems.at[cur],
                    recv_sem=recv_sems.at[nxt],
                    device_id=(right,),
                    device_id_type=pl.DeviceIdType.MESH,
                )
                rdma.start()
            lax.fori_loop(0, n_qblk, qblock, cur)
            if h < N_DEV - 1:
                rdma.wait()
                if h <= N_DEV - 3:
                    pl.semaphore_signal(
                        credit_sem, inc=1,
                        device_id=(left,), device_id_type=pl.DeviceIdType.MESH,
                    )

        out_ref[...] = acc_ref[...] / l_ref[...]

    vmem_limit = 110 * 1024 * 1024
    try:
        params = pltpu.CompilerParams(
            collective_id=0, vmem_limit_bytes=vmem_limit)
    except AttributeError:
        params = pltpu.TPUCompilerParams(
            collective_id=0, vmem_limit_bytes=vmem_limit)

    return pl.pallas_call(
        body,
        out_shape=jax.ShapeDtypeStruct((s_per, d), jnp.float32),
        in_specs=[pl.BlockSpec(memory_space=pltpu.VMEM)] * 3,
        out_specs=pl.BlockSpec(memory_space=pltpu.VMEM),
        scratch_shapes=[
            pltpu.VMEM((2, 2, s_per, d), jnp.bfloat16),
            pltpu.VMEM((s_per, d), jnp.float32),
            pltpu.VMEM((s_per, 1), jnp.float32),
            pltpu.VMEM((s_per, 1), jnp.float32),
            pltpu.SemaphoreType.DMA((2,)),
            pltpu.SemaphoreType.DMA((2,)),
            pltpu.SemaphoreType.REGULAR,
        ],
        compiler_params=params,
    )(q, k, v)


# device time: 938403 ns/iter; 1.6093x vs baseline; 1.6093x over previous
import math

import jax
import jax.numpy as jnp
from jax import lax
from jax.experimental import pallas as pl
from jax.experimental.pallas import tpu as pltpu

N_DEV = 16
QBLK = 512


def kernel(q, k, v):
    s_per, d = q.shape
    s_half = s_per // 2
    scale = 1.0 / math.sqrt(d)
    n_qblk = s_per // QBLK

    q = q.astype(jnp.bfloat16)
    k = k.astype(jnp.bfloat16)
    v = v.astype(jnp.bfloat16)

    def body(q_ref, k_ref, v_ref, out_ref,
             kvr_ref, kvl_ref, acc_ref, m_ref, l_ref,
             send_r, recv_r, send_l, recv_l, credit_r, credit_l):
        my = lax.axis_index("i")
        left = lax.rem(my + N_DEV - 1, N_DEV)
        right = lax.rem(my + 1, N_DEV)

        barrier = pltpu.get_barrier_semaphore()
        for nbr in (left, right):
            pl.semaphore_signal(
                barrier, inc=1,
                device_id=(nbr,), device_id_type=pl.DeviceIdType.MESH,
            )
        pl.semaphore_wait(barrier, 2)

        kvr_ref[0, 0] = k_ref[:s_half, :]
        kvr_ref[0, 1] = v_ref[:s_half, :]
        kvl_ref[0, 0] = k_ref[s_half:, :]
        kvl_ref[0, 1] = v_ref[s_half:, :]
        m_ref[...] = jnp.full((s_per, 1), -jnp.inf, jnp.float32)
        l_ref[...] = jnp.zeros((s_per, 1), jnp.float32)
        acc_ref[...] = jnp.zeros((s_per, d), jnp.float32)

        def make_qblock(kv_ref, cur):
            def qblock(b, carry):
                r = pl.ds(b * QBLK, QBLK)
                qb = q_ref[r, :]
                k_cur = kv_ref[cur, 0]
                v_cur = kv_ref[cur, 1]
                s = lax.dot_general(
                    qb, k_cur, (((1,), (1,)), ((), ())),
                    preferred_element_type=jnp.float32,
                ) * scale
                m_old = m_ref[r, :]
                m_new = jnp.maximum(m_old, jnp.max(s, axis=1, keepdims=True))
                p = jnp.exp(s - m_new)
                corr = jnp.exp(m_old - m_new)
                l_ref[r, :] = l_ref[r, :] * corr + jnp.sum(
                    p, axis=1, keepdims=True)
                acc_ref[r, :] = acc_ref[r, :] * corr + jnp.dot(
                    p.astype(jnp.bfloat16), v_cur,
                    preferred_element_type=jnp.float32,
                )
                m_ref[r, :] = m_new
                return carry
            return qblock

        for h in range(N_DEV):
            cur = h % 2
            nxt = 1 - cur
            if h < N_DEV - 1:
                if h >= 1:
                    pl.semaphore_wait(credit_r, 1)
                    pl.semaphore_wait(credit_l, 1)
                rdma_r = pltpu.make_async_remote_copy(
                    src_ref=kvr_ref.at[cur],
                    dst_ref=kvr_ref.at[nxt],
                    send_sem=send_r.at[cur],
                    recv_sem=recv_r.at[nxt],
                    device_id=(right,),
                    device_id_type=pl.DeviceIdType.MESH,
                )
                rdma_l = pltpu.make_async_remote_copy(
                    src_ref=kvl_ref.at[cur],
                    dst_ref=kvl_ref.at[nxt],
                    send_sem=send_l.at[cur],
                    recv_sem=recv_l.at[nxt],
                    device_id=(left,),
                    device_id_type=pl.DeviceIdType.MESH,
                )
                rdma_r.start()
                rdma_l.start()
            lax.fori_loop(0, n_qblk, make_qblock(kvr_ref, cur), 0)
            lax.fori_loop(0, n_qblk, make_qblock(kvl_ref, cur), 0)
            if h < N_DEV - 1:
                rdma_r.wait()
                rdma_l.wait()
                if h <= N_DEV - 3:
                    pl.semaphore_signal(
                        credit_r, inc=1,
                        device_id=(left,), device_id_type=pl.DeviceIdType.MESH,
                    )
                    pl.semaphore_signal(
                        credit_l, inc=1,
                        device_id=(right,),
                        device_id_type=pl.DeviceIdType.MESH,
                    )

        out_ref[...] = acc_ref[...] / l_ref[...]

    vmem_limit = 110 * 1024 * 1024
    try:
        params = pltpu.CompilerParams(
            collective_id=0, vmem_limit_bytes=vmem_limit)
    except AttributeError:
        params = pltpu.TPUCompilerParams(
            collective_id=0, vmem_limit_bytes=vmem_limit)

    return pl.pallas_call(
        body,
        out_shape=jax.ShapeDtypeStruct((s_per, d), jnp.float32),
        in_specs=[pl.BlockSpec(memory_space=pltpu.VMEM)] * 3,
        out_specs=pl.BlockSpec(memory_space=pltpu.VMEM),
        scratch_shapes=[
            pltpu.VMEM((2, 2, s_half, d), jnp.bfloat16),
            pltpu.VMEM((2, 2, s_half, d), jnp.bfloat16),
            pltpu.VMEM((s_per, d), jnp.float32),
            pltpu.VMEM((s_per, 1), jnp.float32),
            pltpu.VMEM((s_per, 1), jnp.float32),
            pltpu.SemaphoreType.DMA((2,)),
            pltpu.SemaphoreType.DMA((2,)),
            pltpu.SemaphoreType.DMA((2,)),
            pltpu.SemaphoreType.DMA((2,)),
            pltpu.SemaphoreType.REGULAR,
            pltpu.SemaphoreType.REGULAR,
        ],
        compiler_params=params,
    )(q, k, v)


# device time: 885540 ns/iter; 1.7054x vs baseline; 1.0597x over previous
import math

import jax
import jax.numpy as jnp
from jax import lax
from jax.experimental import pallas as pl
from jax.experimental.pallas import tpu as pltpu

N_DEV = 16
QBLK = 512


def kernel(q, k, v):
    s_per, d = q.shape
    s_half = s_per // 2
    scale = 1.0 / math.sqrt(d)
    n_qblk = s_per // QBLK

    q = q.astype(jnp.bfloat16)
    k = k.astype(jnp.bfloat16)
    v = v.astype(jnp.bfloat16)

    def body(q_ref, k_ref, v_ref, out_ref,
             kbuf_ref, vbuf_ref, acc_ref, m_ref, l_ref,
             send_r, recv_r, send_l, recv_l, credit_r, credit_l):
        my = lax.axis_index("i")
        left = lax.rem(my + N_DEV - 1, N_DEV)
        right = lax.rem(my + 1, N_DEV)

        barrier = pltpu.get_barrier_semaphore()
        for nbr in (left, right):
            pl.semaphore_signal(
                barrier, inc=1,
                device_id=(nbr,), device_id_type=pl.DeviceIdType.MESH,
            )
        pl.semaphore_wait(barrier, 2)

        kbuf_ref[0] = k_ref[...]
        vbuf_ref[0] = v_ref[...]
        m_ref[...] = jnp.full((s_per, 1), -jnp.inf, jnp.float32)
        l_ref[...] = jnp.zeros((s_per, 1), jnp.float32)
        acc_ref[...] = jnp.zeros((s_per, d), jnp.float32)

        def qblock(b, carry):
            r = pl.ds(b * QBLK, QBLK)
            cur = carry
            qb = q_ref[r, :]
            s = lax.dot_general(
                qb, kbuf_ref[cur], (((1,), (1,)), ((), ())),
                preferred_element_type=jnp.float32,
            ) * scale
            m_old = m_ref[r, :]
            m_new = jnp.maximum(m_old, jnp.max(s, axis=1, keepdims=True))
            p = jnp.exp(s - m_new)
            corr = jnp.exp(m_old - m_new)
            l_ref[r, :] = l_ref[r, :] * corr + jnp.sum(
                p, axis=1, keepdims=True)
            acc_ref[r, :] = acc_ref[r, :] * corr + jnp.dot(
                p.astype(jnp.bfloat16), vbuf_ref[cur],
                preferred_element_type=jnp.float32,
            )
            m_ref[r, :] = m_new
            return carry

        top = pl.ds(0, s_half)
        bot = pl.ds(s_half, s_half)
        for h in range(N_DEV):
            cur = h % 2
            nxt = 1 - cur
            if h < N_DEV - 1:
                if h >= 1:
                    pl.semaphore_wait(credit_r, 1)
                    pl.semaphore_wait(credit_l, 1)
                rdmas = []
                for buf, kv in ((kbuf_ref, 0), (vbuf_ref, 1)):
                    rdmas.append(pltpu.make_async_remote_copy(
                        src_ref=buf.at[cur, top],
                        dst_ref=buf.at[nxt, top],
                        send_sem=send_r.at[cur, kv],
                        recv_sem=recv_r.at[nxt, kv],
                        device_id=(right,),
                        device_id_type=pl.DeviceIdType.MESH,
                    ))
                    rdmas.append(pltpu.make_async_remote_copy(
                        src_ref=buf.at[cur, bot],
                        dst_ref=buf.at[nxt, bot],
                        send_sem=send_l.at[cur, kv],
                        recv_sem=recv_l.at[nxt, kv],
                        device_id=(left,),
                        device_id_type=pl.DeviceIdType.MESH,
                    ))
                for rdma in rdmas:
                    rdma.start()
            lax.fori_loop(0, n_qblk, qblock, cur)
            if h < N_DEV - 1:
                for rdma in rdmas:
                    rdma.wait()
                if h <= N_DEV - 3:
                    pl.semaphore_signal(
                        credit_r, inc=1,
                        device_id=(left,), device_id_type=pl.DeviceIdType.MESH,
                    )
                    pl.semaphore_signal(
                        credit_l, inc=1,
                        device_id=(right,),
                        device_id_type=pl.DeviceIdType.MESH,
                    )

        out_ref[...] = acc_ref[...] / l_ref[...]

    vmem_limit = 64 * 1024 * 1024 - 64 * 1024
    try:
        params = pltpu.CompilerParams(
            collective_id=0, vmem_limit_bytes=vmem_limit)
    except AttributeError:
        params = pltpu.TPUCompilerParams(
            collective_id=0, vmem_limit_bytes=vmem_limit)

    return pl.pallas_call(
        body,
        out_shape=jax.ShapeDtypeStruct((s_per, d), jnp.float32),
        in_specs=[pl.BlockSpec(memory_space=pltpu.VMEM)] * 3,
        out_specs=pl.BlockSpec(memory_space=pltpu.VMEM),
        scratch_shapes=[
            pltpu.VMEM((2, s_per, d), jnp.bfloat16),
            pltpu.VMEM((2, s_per, d), jnp.bfloat16),
            pltpu.VMEM((s_per, d), jnp.float32),
            pltpu.VMEM((s_per, 1), jnp.float32),
            pltpu.VMEM((s_per, 1), jnp.float32),
            pltpu.SemaphoreType.DMA((2, 2)),
            pltpu.SemaphoreType.DMA((2, 2)),
            pltpu.SemaphoreType.DMA((2, 2)),
            pltpu.SemaphoreType.DMA((2, 2)),
            pltpu.SemaphoreType.REGULAR,
            pltpu.SemaphoreType.REGULAR,
        ],
        compiler_params=params,
    )(q, k, v)
